# baseline (speedup 1.0000x reference)
import jax
import jax.numpy as jnp
from jax import lax
from jax.experimental import pallas as pl
from jax.experimental.pallas import tpu as pltpu
from jax.experimental.pallas import tpu_sc as plsc

N_TOTAL = 8192
C = 512
N_GROUPS = 16
GROUP_N = N_TOTAL // N_GROUPS

_NUM_WORKERS = 32
_ROWS_PER_WORKER = N_TOTAL // _NUM_WORKERS
_CHUNK = 64
_N_CHUNKS = _ROWS_PER_WORKER // _CHUNK
_N_BUFS = 3


def _score_perm_body(x_ref, idx_ref):
    g = pl.program_id(0)
    x = x_ref[...]

    n2 = jnp.sum(x * x, axis=1, keepdims=True)
    norm = jnp.maximum(jnp.sqrt(n2), 1e-12)
    y = x / norm

    sim = lax.dot_general(
        y, y, dimension_numbers=(((1,), (1,)), ((), ())),
        preferred_element_type=jnp.float32,
    )
    scores_col = jnp.sum(sim, axis=1, keepdims=True) / GROUP_N

    n_ids = lax.broadcasted_iota(jnp.int32, (GROUP_N, GROUP_N), 0)
    m_ids = lax.broadcasted_iota(jnp.int32, (GROUP_N, GROUP_N), 1)
    eye = (n_ids == m_ids).astype(jnp.float32)
    scores_row = lax.dot_general(
        scores_col, eye, dimension_numbers=(((0,), (0,)), ((), ())),
        preferred_element_type=jnp.float32,
    )

    gt = scores_row > scores_col
    eq = (scores_row == scores_col) & (m_ids < n_ids)
    contrib = (gt | eq).astype(jnp.float32)
    ones_col = jnp.ones((GROUP_N, 1), jnp.float32)
    rank = lax.dot_general(
        contrib, ones_col, dimension_numbers=(((1,), (0,)), ((), ())),
        preferred_element_type=jnp.float32,
    )

    onehot = (rank == m_ids.astype(jnp.float32)).astype(jnp.float32)
    nvec = n_ids[:, :1].astype(jnp.float32)
    perm = lax.dot_general(
        nvec, onehot, dimension_numbers=(((0,), (0,)), ((), ())),
        preferred_element_type=jnp.float32,
    )

    idx_ref[...] = (perm.astype(jnp.int32) + g * GROUP_N).reshape(1, 1, GROUP_N)


def _sorted_indices(feats):
    return pl.pallas_call(
        _score_perm_body,
        grid=(N_GROUPS,),
        in_specs=[pl.BlockSpec((GROUP_N, C), lambda g: (g, 0))],
        out_specs=pl.BlockSpec((1, 1, GROUP_N), lambda g: (g, 0, 0)),
        out_shape=jax.ShapeDtypeStruct((N_GROUPS, 1, GROUP_N), jnp.int32),
    )(feats)


def _input_writer_body(x_ref, out_ref):
    g = pl.program_id(0)
    out_ref[pl.ds(g, 1), :] = x_ref[...].reshape(1, GROUP_N * C)


def _write_out_input(feats):
    return pl.pallas_call(
        _input_writer_body,
        grid=(N_GROUPS,),
        in_specs=[pl.BlockSpec((GROUP_N, C), lambda g: (g, 0))],
        out_specs=pl.BlockSpec((N_GROUPS, GROUP_N * C), lambda g: (0, 0)),
        out_shape=jax.ShapeDtypeStruct((N_GROUPS, GROUP_N * C), jnp.float32),
    )(feats)


def _gather_body(feats_hbm, idx_hbm, out_hbm, idx_v, rows0, rows1, rows2,
                 gsem0, gsem1, gsem2, wsem0, wsem1, wsem2):
    wid = lax.axis_index("s") * 2 + lax.axis_index("c")
    g = wid // 2
    half = wid % 2
    base = wid * _ROWS_PER_WORKER
    pltpu.sync_copy(idx_hbm.at[pl.ds(base, _ROWS_PER_WORKER)], idx_v)
    bufs = (rows0, rows1, rows2)
    gsems = (gsem0, gsem1, gsem2)
    wsems = (wsem0, wsem1, wsem2)
    gathers = [None] * _N_CHUNKS
    for c in range(_N_BUFS):
        gathers[c] = pltpu.async_copy(
            feats_hbm.at[idx_v.at[pl.ds(c * _CHUNK, _CHUNK)]], bufs[c], gsems[c])
    for c in range(_N_CHUNKS):
        b = c % _N_BUFS
        buf, ws = bufs[b], wsems[b]
        gathers[c].wait()
        col0 = (half * _ROWS_PER_WORKER + c * _CHUNK) * C

        @pl.loop(0, _CHUNK, unroll=8)
        def _row_write(r):
            pltpu.async_copy(
                buf.at[pl.ds(r, 1), :],
                out_hbm.at[pl.ds(g, 1), pl.ds(col0 + r * C, C)],
                ws)

        if c + _N_BUFS < _N_CHUNKS:
            pltpu.make_async_copy(feats_hbm.at[pl.ds(0, _CHUNK)], buf, ws).wait()
            gathers[c + _N_BUFS] = pltpu.async_copy(
                feats_hbm.at[idx_v.at[pl.ds((c + _N_BUFS) * _CHUNK, _CHUNK)]],
                buf, gsems[b])
    for c in range(max(0, _N_CHUNKS - _N_BUFS), _N_CHUNKS):
        b = c % _N_BUFS
        pltpu.make_async_copy(
            feats_hbm.at[pl.ds(0, _CHUNK)], bufs[b], wsems[b]).wait()


def _gather_rows(feats, idx):
    gather = pl.kernel(
        _gather_body,
        out_type=jax.ShapeDtypeStruct((N_GROUPS, GROUP_N * C), jnp.float32),
        scratch_types=[
            pltpu.VMEM((_ROWS_PER_WORKER,), jnp.int32),
            pltpu.VMEM((_CHUNK, C), jnp.float32),
            pltpu.VMEM((_CHUNK, C), jnp.float32),
            pltpu.VMEM((_CHUNK, C), jnp.float32),
            pltpu.SemaphoreType.DMA,
            pltpu.SemaphoreType.DMA,
            pltpu.SemaphoreType.DMA,
            pltpu.SemaphoreType.DMA,
            pltpu.SemaphoreType.DMA,
            pltpu.SemaphoreType.DMA,
        ],
        mesh=plsc.VectorSubcoreMesh(core_axis_name="c", subcore_axis_name="s"),
    )
    return gather(feats, idx)


def kernel(feats, labels, training):
    del labels, training
    idx = _sorted_indices(feats)
    out_sorted = _gather_rows(feats, idx.reshape(N_TOTAL))
    out_input = _write_out_input(feats)
    return (out_sorted, out_input)


# speedup vs baseline: 1.0756x; 1.0756x over previous
import jax
import jax.numpy as jnp
from jax import lax
from jax.experimental import pallas as pl
from jax.experimental.pallas import tpu as pltpu
from jax.experimental.pallas import tpu_sc as plsc

N_TOTAL = 8192
C = 512
N_GROUPS = 16
GROUP_N = N_TOTAL // N_GROUPS

_NUM_WORKERS = 32
_ROWS_PER_WORKER = N_TOTAL // _NUM_WORKERS
_CHUNK = 64
_N_CHUNKS = _ROWS_PER_WORKER // _CHUNK
_N_BUFS = 3


def _score_perm_body(x_ref, idx_ref):
    g = pl.program_id(0)
    x = x_ref[...]

    n2 = jnp.sum(x * x, axis=1, keepdims=True)
    norm = jnp.maximum(jnp.sqrt(n2), 1e-12)
    y = x / norm

    sim = lax.dot_general(
        y, y, dimension_numbers=(((1,), (1,)), ((), ())),
        preferred_element_type=jnp.float32,
    )
    scores_col = jnp.sum(sim, axis=1, keepdims=True) / GROUP_N

    n_ids = lax.broadcasted_iota(jnp.int32, (GROUP_N, GROUP_N), 0)
    m_ids = lax.broadcasted_iota(jnp.int32, (GROUP_N, GROUP_N), 1)
    eye = (n_ids == m_ids).astype(jnp.float32)
    scores_row = lax.dot_general(
        scores_col, eye, dimension_numbers=(((0,), (0,)), ((), ())),
        preferred_element_type=jnp.float32,
    )

    gt = scores_row > scores_col
    eq = (scores_row == scores_col) & (m_ids < n_ids)
    rank = jnp.sum((gt | eq).astype(jnp.int32), axis=1, keepdims=True)

    onehot = rank == m_ids
    perm = jnp.sum(jnp.where(onehot, n_ids, 0), axis=0, keepdims=True)

    idx_ref[...] = (perm + g * GROUP_N).reshape(1, 1, GROUP_N)


def _sorted_indices(feats):
    return pl.pallas_call(
        _score_perm_body,
        grid=(N_GROUPS,),
        in_specs=[pl.BlockSpec((GROUP_N, C), lambda g: (g, 0))],
        out_specs=pl.BlockSpec((1, 1, GROUP_N), lambda g: (g, 0, 0)),
        out_shape=jax.ShapeDtypeStruct((N_GROUPS, 1, GROUP_N), jnp.int32),
    )(feats)


def _input_writer_body(x_ref, out_ref):
    g = pl.program_id(0)
    out_ref[pl.ds(g, 1), :] = x_ref[...].reshape(1, GROUP_N * C)


def _write_out_input(feats):
    return pl.pallas_call(
        _input_writer_body,
        grid=(N_GROUPS,),
        in_specs=[pl.BlockSpec((GROUP_N, C), lambda g: (g, 0))],
        out_specs=pl.BlockSpec((N_GROUPS, GROUP_N * C), lambda g: (0, 0)),
        out_shape=jax.ShapeDtypeStruct((N_GROUPS, GROUP_N * C), jnp.float32),
    )(feats)


def _gather_body(feats_hbm, idx_hbm, out_hbm, idx_v, rows0, rows1, rows2,
                 gsem0, gsem1, gsem2, wsem0, wsem1, wsem2):
    wid = lax.axis_index("s") * 2 + lax.axis_index("c")
    g = wid // 2
    half = wid % 2
    base = wid * _ROWS_PER_WORKER
    pltpu.sync_copy(idx_hbm.at[pl.ds(base, _ROWS_PER_WORKER)], idx_v)
    bufs = (rows0, rows1, rows2)
    gsems = (gsem0, gsem1, gsem2)
    wsems = (wsem0, wsem1, wsem2)
    gathers = [None] * _N_CHUNKS
    for c in range(_N_BUFS):
        gathers[c] = pltpu.async_copy(
            feats_hbm.at[idx_v.at[pl.ds(c * _CHUNK, _CHUNK)]], bufs[c], gsems[c])
    for c in range(_N_CHUNKS):
        b = c % _N_BUFS
        buf, ws = bufs[b], wsems[b]
        gathers[c].wait()
        col0 = (half * _ROWS_PER_WORKER + c * _CHUNK) * C

        @pl.loop(0, _CHUNK, unroll=8)
        def _row_write(r):
            pltpu.async_copy(
                buf.at[pl.ds(r, 1), :],
                out_hbm.at[pl.ds(g, 1), pl.ds(col0 + r * C, C)],
                ws)

        if c + _N_BUFS < _N_CHUNKS:
            pltpu.make_async_copy(feats_hbm.at[pl.ds(0, _CHUNK)], buf, ws).wait()
            gathers[c + _N_BUFS] = pltpu.async_copy(
                feats_hbm.at[idx_v.at[pl.ds((c + _N_BUFS) * _CHUNK, _CHUNK)]],
                buf, gsems[b])
    for c in range(max(0, _N_CHUNKS - _N_BUFS), _N_CHUNKS):
        b = c % _N_BUFS
        pltpu.make_async_copy(
            feats_hbm.at[pl.ds(0, _CHUNK)], bufs[b], wsems[b]).wait()


def _gather_rows(feats, idx):
    gather = pl.kernel(
        _gather_body,
        out_type=jax.ShapeDtypeStruct((N_GROUPS, GROUP_N * C), jnp.float32),
        scratch_types=[
            pltpu.VMEM((_ROWS_PER_WORKER,), jnp.int32),
            pltpu.VMEM((_CHUNK, C), jnp.float32),
            pltpu.VMEM((_CHUNK, C), jnp.float32),
            pltpu.VMEM((_CHUNK, C), jnp.float32),
            pltpu.SemaphoreType.DMA,
            pltpu.SemaphoreType.DMA,
            pltpu.SemaphoreType.DMA,
            pltpu.SemaphoreType.DMA,
            pltpu.SemaphoreType.DMA,
            pltpu.SemaphoreType.DMA,
        ],
        mesh=plsc.VectorSubcoreMesh(core_axis_name="c", subcore_axis_name="s"),
    )
    return gather(feats, idx)


def kernel(feats, labels, training):
    del labels, training
    idx = _sorted_indices(feats)
    out_sorted = _gather_rows(feats, idx.reshape(N_TOTAL))
    out_input = _write_out_input(feats)
    return (out_sorted, out_input)
